# baseline (device time: 21228 ns/iter reference)
import jax
import jax.numpy as jnp
from jax import lax
from jax.experimental import pallas as pl
from jax.experimental.pallas import tpu as pltpu

Y_DEV = 4
BM = 512


def _peers(my_x, my_y, my_z):
    return [
        (my_x, (my_y + k) % Y_DEV, my_z) for k in range(1, Y_DEV)
    ]


def _block_partial(x, dy):
    mu = jnp.mean(x, axis=1, keepdims=True)
    var = jnp.mean(x * x, axis=1, keepdims=True) - mu * mu
    rstd = lax.rsqrt(var + 1e-5)
    xhat = (x - mu) * rstd
    dg = jnp.sum(dy * xhat, axis=0, keepdims=True)
    db = jnp.sum(dy, axis=0, keepdims=True)
    return jnp.concatenate([dg, db], axis=0)


def _body(x_ref, dy_ref, out_ref, comm_ref, send_sems, recv_sems):
    step = pl.program_id(0)
    my_x = lax.axis_index("x")
    my_y = lax.axis_index("y")
    my_z = lax.axis_index("z")
    peers = _peers(my_x, my_y, my_z)

    def push_half(half):
        rdmas = []
        for k in range(1, Y_DEV):
            rdma = pltpu.make_async_remote_copy(
                src_ref=comm_ref.at[half, 0],
                dst_ref=comm_ref.at[half, k],
                send_sem=send_sems.at[half, k - 1],
                recv_sem=recv_sems.at[half, k - 1],
                device_id=peers[k - 1],
                device_id_type=pl.DeviceIdType.MESH,
            )
            rdma.start()
            rdmas.append(rdma)
        return rdmas

    @pl.when(step == 0)
    def _():
        barrier_sem = pltpu.get_barrier_semaphore()
        for p in peers:
            pl.semaphore_signal(
                barrier_sem, inc=1, device_id=p,
                device_id_type=pl.DeviceIdType.MESH,
            )

    partial = _block_partial(x_ref[:, :], dy_ref[:, :])

    @pl.when(step == 0)
    def _():
        comm_ref[0, 0, :, :] = partial

    @pl.when(step == 1)
    def _():
        comm_ref[0, 0, :, :] = comm_ref[0, 0, :, :] + partial
        pl.semaphore_wait(pltpu.get_barrier_semaphore(), Y_DEV - 1)
        push_half(0)

    @pl.when(step == 2)
    def _():
        comm_ref[1, 0, :, :] = partial

    @pl.when(step == 3)
    def _():
        comm_ref[1, 0, :, :] = comm_ref[1, 0, :, :] + partial
        rdmas_b = push_half(1)
        for k in range(1, Y_DEV):
            rdma_a = pltpu.make_async_remote_copy(
                src_ref=comm_ref.at[0, 0],
                dst_ref=comm_ref.at[0, k],
                send_sem=send_sems.at[0, k - 1],
                recv_sem=recv_sems.at[0, k - 1],
                device_id=peers[k - 1],
                device_id_type=pl.DeviceIdType.MESH,
            )
            rdma_a.wait()
        for rdma in rdmas_b:
            rdma.wait()
        acc = comm_ref[0, 0, :, :]
        for half in range(2):
            for slot in range(Y_DEV):
                if half == 0 and slot == 0:
                    continue
                acc = acc + comm_ref[half, slot, :, :]
        out_ref[:, :] = acc


def kernel(x, dy, gamma):
    del gamma
    m, d = x.shape
    num_blocks = m // BM
    assert num_blocks == 4

    return pl.pallas_call(
        _body,
        grid=(num_blocks,),
        in_specs=[
            pl.BlockSpec((BM, d), lambda i: (i, 0)),
            pl.BlockSpec((BM, d), lambda i: (i, 0)),
        ],
        out_specs=pl.BlockSpec((2, d), lambda i: (0, 0)),
        out_shape=jax.ShapeDtypeStruct((2, d), jnp.float32),
        scratch_shapes=[
            pltpu.VMEM((2, Y_DEV, 2, d), jnp.float32),
            pltpu.SemaphoreType.DMA((2, Y_DEV - 1)),
            pltpu.SemaphoreType.DMA((2, Y_DEV - 1)),
        ],
        compiler_params=pltpu.CompilerParams(collective_id=0),
    )(x, dy)
